# baseline (speedup 1.0000x reference)
import functools

import jax
import jax.numpy as jnp
from jax.experimental import pallas as pl
from jax.experimental.pallas import tpu as pltpu

IN_F = 128
HID_F = 128
OUT_F = 128
N_NGBRS = 320000

BLOCK = 4000
NUM_BLOCKS = N_NGBRS // BLOCK

_SLOPE = 0.01


def _lrelu(x):
    return jnp.where(x >= 0, x, _SLOPE * x)


def _pinsage_kernel(x_ref, a_ref, qt_ref, qb_ref, hn_ref, wt_ref, wb_ref,
                    out_ref, acc_ref, asum_ref):
    i = pl.program_id(0)

    @pl.when(i == 0)
    def _():
        acc_ref[...] = jnp.zeros_like(acc_ref)
        asum_ref[0, 0] = 0.0

    x = x_ref[...]
    a = a_ref[...].reshape(1, BLOCK)

    z = jax.lax.dot_general(
        x, qt_ref[...], (((1,), (0,)), ((), ())),
        preferred_element_type=jnp.float32,
        precision=jax.lax.Precision.HIGHEST,
    ) + qb_ref[...]
    l = _lrelu(z)

    partial = jax.lax.dot_general(
        a, l, (((1,), (0,)), ((), ())),
        preferred_element_type=jnp.float32,
        precision=jax.lax.Precision.HIGHEST,
    )
    acc_ref[...] += partial
    asum_ref[0, 0] += jnp.sum(a)

    @pl.when(i == NUM_BLOCKS - 1)
    def _():
        asum = asum_ref[0, 0]
        asafe = jnp.where(asum == 0.0, 1.0, asum)
        h_agg = acc_ref[...] / asafe

        wt = wt_ref[...]
        z2 = jax.lax.dot_general(
            hn_ref[...], wt[:IN_F, :], (((1,), (0,)), ((), ())),
            preferred_element_type=jnp.float32,
            precision=jax.lax.Precision.HIGHEST,
        ) + jax.lax.dot_general(
            h_agg, wt[IN_F:, :], (((1,), (0,)), ((), ())),
            preferred_element_type=jnp.float32,
            precision=jax.lax.Precision.HIGHEST,
        ) + wb_ref[...]
        h_two = _lrelu(z2)
        nrm = jnp.sqrt(jnp.sum(h_two * h_two))
        nsafe = jnp.where(nrm == 0.0, 1.0, nrm)
        out_ref[...] = h_two / nsafe


@jax.jit
def kernel(h_node, h_ngbrs, alpha, Q_w, Q_b, W_w, W_b):
    alpha3 = alpha.reshape(NUM_BLOCKS, 1, BLOCK)
    out = pl.pallas_call(
        _pinsage_kernel,
        grid=(NUM_BLOCKS,),
        in_specs=[
            pl.BlockSpec((BLOCK, IN_F), lambda i: (i, 0)),
            pl.BlockSpec((1, 1, BLOCK), lambda i: (i, 0, 0)),
            pl.BlockSpec((IN_F, HID_F), lambda i: (0, 0)),
            pl.BlockSpec((1, HID_F), lambda i: (0, 0)),
            pl.BlockSpec((1, IN_F), lambda i: (0, 0)),
            pl.BlockSpec((IN_F + HID_F, OUT_F), lambda i: (0, 0)),
            pl.BlockSpec((1, OUT_F), lambda i: (0, 0)),
        ],
        out_specs=pl.BlockSpec((1, OUT_F), lambda i: (0, 0)),
        out_shape=jax.ShapeDtypeStruct((1, OUT_F), jnp.float32),
        scratch_shapes=[
            pltpu.VMEM((1, HID_F), jnp.float32),
            pltpu.SMEM((1, 1), jnp.float32),
        ],
    )(
        h_ngbrs,
        alpha3,
        Q_w.T,
        Q_b.reshape(1, HID_F),
        h_node.reshape(1, IN_F),
        W_w.T,
        W_b.reshape(1, OUT_F),
    )
    return out[0]


# speedup vs baseline: 1.7213x; 1.7213x over previous
import jax
import jax.numpy as jnp
from jax.experimental import pallas as pl
from jax.experimental.pallas import tpu as pltpu

IN_F = 128
HID_F = 128
OUT_F = 128
N_NGBRS = 320000

BLOCK = 4000
NUM_BLOCKS = N_NGBRS // BLOCK

_SLOPE = 0.01


def _lrelu(x):
    return jnp.where(x >= 0, x, _SLOPE * x)


def _pinsage_kernel(x_ref, a_ref, qt_ref, qb_ref, hn_ref, wt_ref, wb_ref,
                    out_ref, acc_ref, asum_ref):
    i = pl.program_id(0)

    @pl.when(i == 0)
    def _():
        acc_ref[...] = jnp.zeros_like(acc_ref)
        asum_ref[0, 0] = 0.0

    x = x_ref[...]
    a = a_ref[...]

    z = jax.lax.dot_general(
        x, qt_ref[...], (((1,), (0,)), ((), ())),
        preferred_element_type=jnp.float32,
    ) + qb_ref[...]
    weighted = _lrelu(z) * a
    acc_ref[...] += jnp.sum(weighted, axis=0, keepdims=True)
    asum_ref[0, 0] += jnp.sum(a)

    @pl.when(i == NUM_BLOCKS - 1)
    def _():
        asum = asum_ref[0, 0]
        asafe = jnp.where(asum == 0.0, 1.0, asum)
        h_agg = acc_ref[...] / asafe

        wt = wt_ref[...]
        z2 = jax.lax.dot_general(
            hn_ref[...], wt[:IN_F, :], (((1,), (0,)), ((), ())),
            preferred_element_type=jnp.float32,
        ) + jax.lax.dot_general(
            h_agg, wt[IN_F:, :], (((1,), (0,)), ((), ())),
            preferred_element_type=jnp.float32,
        ) + wb_ref[...]
        h_two = _lrelu(z2)
        nrm = jnp.sqrt(jnp.sum(h_two * h_two))
        nsafe = jnp.where(nrm == 0.0, 1.0, nrm)
        out_ref[...] = h_two / nsafe


@jax.jit
def kernel(h_node, h_ngbrs, alpha, Q_w, Q_b, W_w, W_b):
    out = pl.pallas_call(
        _pinsage_kernel,
        grid=(NUM_BLOCKS,),
        in_specs=[
            pl.BlockSpec((BLOCK, IN_F), lambda i: (i, 0)),
            pl.BlockSpec((BLOCK, 1), lambda i: (i, 0)),
            pl.BlockSpec((IN_F, HID_F), lambda i: (0, 0)),
            pl.BlockSpec((1, HID_F), lambda i: (0, 0)),
            pl.BlockSpec((1, IN_F), lambda i: (0, 0)),
            pl.BlockSpec((IN_F + HID_F, OUT_F), lambda i: (0, 0)),
            pl.BlockSpec((1, OUT_F), lambda i: (0, 0)),
        ],
        out_specs=pl.BlockSpec((1, OUT_F), lambda i: (0, 0)),
        out_shape=jax.ShapeDtypeStruct((1, OUT_F), jnp.float32),
        scratch_shapes=[
            pltpu.VMEM((1, HID_F), jnp.float32),
            pltpu.SMEM((1, 1), jnp.float32),
        ],
    )(
        h_ngbrs,
        alpha,
        Q_w.T,
        Q_b.reshape(1, HID_F),
        h_node.reshape(1, IN_F),
        W_w.T,
        W_b.reshape(1, OUT_F),
    )
    return out[0]


# speedup vs baseline: 3.4866x; 2.0256x over previous
import jax
import jax.numpy as jnp
from jax.experimental import pallas as pl
from jax.experimental.pallas import tpu as pltpu

IN_F = 128
HID_F = 128
OUT_F = 128
N_NGBRS = 320000

K_STREAMS = 4
BLOCK = 2000
NUM_STEPS = N_NGBRS // (K_STREAMS * BLOCK)

_SLOPE = 0.01


def _lrelu(x):
    return jnp.maximum(x, _SLOPE * x)


def _dot(a, b):
    return jax.lax.dot_general(
        a, b, (((1,), (0,)), ((), ())), preferred_element_type=jnp.float32)


def _pinsage_kernel(*refs):
    x_refs = refs[:K_STREAMS]
    a_refs = refs[K_STREAMS:2 * K_STREAMS]
    qt_ref, qb_ref, hn_ref, wt_ref, wb_ref, out_ref, acc_ref, asum_ref = \
        refs[2 * K_STREAMS:]
    i = pl.program_id(0)

    @pl.when(i == 0)
    def _():
        acc_ref[...] = jnp.zeros_like(acc_ref)
        asum_ref[0, 0] = 0.0

    qt = qt_ref[...]
    qb = qb_ref[...]
    acc = acc_ref[...]
    asum = asum_ref[0, 0]
    for k in range(K_STREAMS):
        a = a_refs[k][...].reshape(1, BLOCK)
        l = _lrelu(_dot(x_refs[k][...], qt) + qb)
        acc = acc + _dot(a, l)
        asum = asum + jnp.sum(a)
    acc_ref[...] = acc
    asum_ref[0, 0] = asum

    @pl.when(i == NUM_STEPS - 1)
    def _():
        s = asum_ref[0, 0]
        ssafe = jnp.where(s == 0.0, 1.0, s)
        h_agg = acc_ref[...] / ssafe

        wt = wt_ref[...]
        z2 = _dot(hn_ref[...], wt[:IN_F, :]) + _dot(h_agg, wt[IN_F:, :]) \
            + wb_ref[...]
        h_two = _lrelu(z2)
        nrm = jnp.sqrt(jnp.sum(h_two * h_two))
        nsafe = jnp.where(nrm == 0.0, 1.0, nrm)
        out_ref[...] = h_two / nsafe


@jax.jit
def kernel(h_node, h_ngbrs, alpha, Q_w, Q_b, W_w, W_b):
    alpha4 = alpha.reshape(K_STREAMS, NUM_STEPS, 1, BLOCK)

    def x_spec(k):
        return pl.BlockSpec((BLOCK, IN_F), lambda i, k=k: (k * NUM_STEPS + i, 0))

    in_specs = [x_spec(k) for k in range(K_STREAMS)]
    a_specs = [
        pl.BlockSpec((1, 1, 1, BLOCK), lambda i, k=k: (k, i, 0, 0))
        for k in range(K_STREAMS)
    ]
    out = pl.pallas_call(
        _pinsage_kernel,
        grid=(NUM_STEPS,),
        in_specs=in_specs + a_specs + [
            pl.BlockSpec((IN_F, HID_F), lambda i: (0, 0)),
            pl.BlockSpec((1, HID_F), lambda i: (0, 0)),
            pl.BlockSpec((1, IN_F), lambda i: (0, 0)),
            pl.BlockSpec((IN_F + HID_F, OUT_F), lambda i: (0, 0)),
            pl.BlockSpec((1, OUT_F), lambda i: (0, 0)),
        ],
        out_specs=pl.BlockSpec((1, OUT_F), lambda i: (0, 0)),
        out_shape=jax.ShapeDtypeStruct((1, OUT_F), jnp.float32),
        scratch_shapes=[
            pltpu.VMEM((1, HID_F), jnp.float32),
            pltpu.SMEM((1, 1), jnp.float32),
        ],
    )(
        *([h_ngbrs] * K_STREAMS),
        *([alpha4] * K_STREAMS),
        Q_w.T,
        Q_b.reshape(1, HID_F),
        h_node.reshape(1, IN_F),
        W_w.T,
        W_b.reshape(1, OUT_F),
    )
    return out[0]
